# Initial kernel scaffold; baseline (speedup 1.0000x reference)
import jax
import jax.numpy as jnp
from jax.experimental import pallas as pl


def kernel(node_features, edge_index, edge_attr, post_mask, W_node, b_node, W_edge, b_edge, W_conv, b_conv, W_out, b_out):
    raise NotImplementedError("write your pallas kernel here")


# speedup vs baseline: 3.4644x; 3.4644x over previous
import functools

import jax
import jax.numpy as jnp
from jax import lax
from jax.experimental import pallas as pl
from jax.experimental.pallas import tpu as pltpu
from jax.experimental.pallas import tpu_sc as plsc

N = 10000
E = 320000
D_NODE = 128
D_EDGE = 16
H = 32
P = 5000

NC = 2
NS = 16
NW = NC * NS
S = 128
NCH = E // S
FULL_ROUNDS = NCH // NW
LEFTOVER = NCH - FULL_ROUNDS * NW
P_PAD = 5120
NCHUNK_P = P_PAD // S

STRIPE = 624
LAST_STRIPE = 640

_EDGE_BLK = 16000
_mesh = plsc.VectorSubcoreMesh(core_axis_name="c", subcore_axis_name="s")


def _tc_nodes_body(nf_ref, wnt_ref, bn_ref, wc1t_ref, ne_ref, a_ref):
    ne = jnp.maximum(
        jnp.dot(nf_ref[...], wnt_ref[...], preferred_element_type=jnp.float32)
        + bn_ref[...], 0.0)
    ne_ref[...] = ne
    a_ref[...] = jnp.dot(ne, wc1t_ref[...], preferred_element_type=jnp.float32)


_tc_nodes = pl.pallas_call(
    _tc_nodes_body,
    out_shape=(jax.ShapeDtypeStruct((N, H), jnp.float32),
               jax.ShapeDtypeStruct((N, H), jnp.float32)),
)


def _tc_edges_body(ea_ref, wet_ref, be_ref, wc2t_ref, bc_ref, b_ref):
    ee = jnp.maximum(
        jnp.dot(ea_ref[...], wet_ref[...], preferred_element_type=jnp.float32)
        + be_ref[...], 0.0)
    b_ref[...] = (
        jnp.dot(ee, wc2t_ref[...], preferred_element_type=jnp.float32)
        + bc_ref[...])


_tc_edges = pl.pallas_call(
    _tc_edges_body,
    grid=(E // _EDGE_BLK,),
    in_specs=[
        pl.BlockSpec((_EDGE_BLK, D_EDGE), lambda i: (i, 0)),
        pl.BlockSpec((D_EDGE, H), lambda i: (0, 0)),
        pl.BlockSpec((1, H), lambda i: (0, 0)),
        pl.BlockSpec((H, H), lambda i: (0, 0)),
        pl.BlockSpec((1, H), lambda i: (0, 0)),
    ],
    out_specs=pl.BlockSpec((_EDGE_BLK, H), lambda i: (i, 0)),
    out_shape=jax.ShapeDtypeStruct((E, H), jnp.float32),
)


def _tc_combine_body(ne_ref, p_ref, y_ref):
    y = ne_ref[...] + p_ref[0] + p_ref[1]
    y_ref[...] = jnp.concatenate(
        [y, jnp.zeros((N, 128 - H), jnp.float32)], axis=1)


_tc_combine = pl.pallas_call(
    _tc_combine_body,
    out_shape=jax.ShapeDtypeStruct((N, 128), jnp.float32),
)


def _tc_head_body(post_ref, wot_ref, bo_ref, out_ref):
    logits = (jnp.dot(post_ref[...], wot_ref[...],
                      preferred_element_type=jnp.float32) + bo_ref[...])
    out_ref[...] = jax.nn.sigmoid(logits)


_tc_head = pl.pallas_call(
    _tc_head_body,
    out_shape=jax.ShapeDtypeStruct((P, 1), jnp.float32),
)


@functools.partial(
    pl.kernel,
    out_type=jax.ShapeDtypeStruct((NC, N, H), jnp.float32),
    mesh=_mesh,
    compiler_params=pltpu.CompilerParams(use_tc_tiling_on_sc=False),
    scratch_types=[
        pltpu.VMEM((S,), jnp.int32),
        pltpu.VMEM((S,), jnp.int32),
        pltpu.VMEM((S, H), jnp.float32),
        pltpu.VMEM((S, H), jnp.float32),
        pltpu.VMEM((LAST_STRIPE, H), jnp.float32),
        pltpu.VMEM_SHARED((N, H), jnp.float32),
        pltpu.VMEM_SHARED((N, H), jnp.float32),
        pltpu.SemaphoreType.DMA,
    ],
)
def _sc_messages(a_hbm, b_hbm, src_hbm, dst_hbm, out_hbm,
                 src_v, dst_v, arows_v, brows_v, zbuf_v, msg_sh, a_sh, sem):
    cid = lax.axis_index("c")
    sid = lax.axis_index("s")
    wid = sid * NC + cid

    @pl.when(sid < NS - 1)
    def _a_main():
        off = pl.multiple_of(sid * STRIPE, 8)
        pltpu.sync_copy(a_hbm.at[pl.ds(off, STRIPE), :],
                        a_sh.at[pl.ds(off, STRIPE), :])

    @pl.when(sid == NS - 1)
    def _a_last():
        off = (NS - 1) * STRIPE
        pltpu.sync_copy(a_hbm.at[pl.ds(off, LAST_STRIPE), :],
                        a_sh.at[pl.ds(off, LAST_STRIPE), :])

    def _zb(i, carry):
        zbuf_v[i, pl.ds(0, 16)] = jnp.zeros((16,), jnp.float32)
        zbuf_v[i, pl.ds(16, 16)] = jnp.zeros((16,), jnp.float32)
        return carry

    lax.fori_loop(0, LAST_STRIPE, _zb, 0)

    @pl.when(sid < NS - 1)
    def _z_main():
        off = pl.multiple_of(sid * STRIPE, 8)
        pltpu.sync_copy(zbuf_v.at[pl.ds(0, STRIPE), :],
                        msg_sh.at[pl.ds(off, STRIPE), :])

    @pl.when(sid == NS - 1)
    def _z_last():
        pltpu.sync_copy(zbuf_v, msg_sh.at[pl.ds((NS - 1) * STRIPE,
                                                LAST_STRIPE), :])

    plsc.subcore_barrier()

    def _process(c):
        eb = pl.multiple_of(c * S, 8)
        pltpu.sync_copy(src_hbm.at[pl.ds(eb, S)], src_v)
        pltpu.sync_copy(dst_hbm.at[pl.ds(eb, S)], dst_v)
        pltpu.sync_copy(b_hbm.at[pl.ds(eb, S), :], brows_v)
        pltpu.async_copy(a_sh.at[src_v], arows_v, sem).wait()
        pltpu.sync_copy(arows_v, msg_sh.at[dst_v], add=True)
        pltpu.sync_copy(brows_v, msg_sh.at[dst_v], add=True)

    def _body(j, carry):
        _process(wid + j * NW)
        return carry

    lax.fori_loop(0, FULL_ROUNDS, _body, 0)

    @pl.when(wid < LEFTOVER)
    def _tail():
        _process(FULL_ROUNDS * NW + wid)

    plsc.subcore_barrier()

    @pl.when(sid < NS - 1)
    def _d_main():
        off = pl.multiple_of(sid * STRIPE, 8)
        pltpu.sync_copy(msg_sh.at[pl.ds(off, STRIPE), :],
                        out_hbm.at[cid, pl.ds(off, STRIPE), :])

    @pl.when(sid == NS - 1)
    def _d_last():
        pltpu.sync_copy(msg_sh.at[pl.ds((NS - 1) * STRIPE, LAST_STRIPE), :],
                        out_hbm.at[cid, pl.ds((NS - 1) * STRIPE,
                                              LAST_STRIPE), :])


@functools.partial(
    pl.kernel,
    out_type=jax.ShapeDtypeStruct((P_PAD, 128), jnp.float32),
    mesh=_mesh,
    compiler_params=pltpu.CompilerParams(use_tc_tiling_on_sc=False),
    scratch_types=[
        pltpu.VMEM((S,), jnp.int32),
        pltpu.VMEM((S, 128), jnp.float32),
        pltpu.SemaphoreType.DMA,
    ],
)
def _sc_post_gather(y_hbm, pm_hbm, out_hbm, pm_v, rows_v, sem):
    cid = lax.axis_index("c")
    sid = lax.axis_index("s")
    wid = sid * NC + cid

    def _do(chunk):
        off = pl.multiple_of(chunk * S, 8)
        pltpu.sync_copy(pm_hbm.at[pl.ds(off, S)], pm_v)
        pltpu.async_copy(y_hbm.at[pm_v], rows_v, sem).wait()
        pltpu.sync_copy(rows_v, out_hbm.at[pl.ds(off, S), :])

    _do(wid)

    @pl.when(wid < NCHUNK_P - NW)
    def _tail():
        _do(wid + NW)


def kernel(node_features, edge_index, edge_attr, post_mask,
           W_node, b_node, W_edge, b_edge, W_conv, b_conv, W_out, b_out):
    wnt = W_node.T
    wet = W_edge.T
    wc1t = W_conv[:, :H].T
    wc2t = W_conv[:, H:].T
    wot = W_out.T
    bn = b_node.reshape(1, H)
    be = b_edge.reshape(1, H)
    bc = b_conv.reshape(1, H)
    bo = b_out.reshape(1, 1)

    node_emb, a_rows = _tc_nodes(node_features, wnt, bn, wc1t)
    b_rows = _tc_edges(edge_attr, wet, be, wc2t, bc)

    partials = _sc_messages(a_rows, b_rows, edge_index[0], edge_index[1])

    y = _tc_combine(node_emb, partials)
    pm_pad = jnp.concatenate(
        [post_mask, jnp.zeros((P_PAD - P,), dtype=post_mask.dtype)])
    post_full = _sc_post_gather(y, pm_pad)
    out = _tc_head(post_full[:P, :H], wot, bo)
    return out.reshape(P)


# speedup vs baseline: 4.4394x; 1.2814x over previous
import functools

import jax
import jax.numpy as jnp
from jax import lax
from jax.experimental import pallas as pl
from jax.experimental.pallas import tpu as pltpu
from jax.experimental.pallas import tpu_sc as plsc

N = 10000
E = 320000
D_NODE = 128
D_EDGE = 16
H = 32
P = 5000

NC = 2
NS = 16
NW = NC * NS
S = 128
NCH = E // S
FULL_ROUNDS = NCH // NW
LEFTOVER = NCH - FULL_ROUNDS * NW
P_PAD = 5120
NCHUNK_P = P_PAD // S

STRIPE = 624
LAST_STRIPE = 640

_EDGE_BLK = 16000
_mesh = plsc.VectorSubcoreMesh(core_axis_name="c", subcore_axis_name="s")


def _tc_nodes_body(nf_ref, wnt_ref, bn_ref, wc1t_ref, ne_ref, a_ref):
    ne = jnp.maximum(
        jnp.dot(nf_ref[...], wnt_ref[...], preferred_element_type=jnp.float32)
        + bn_ref[...], 0.0)
    ne_ref[...] = ne
    a_ref[...] = jnp.dot(ne, wc1t_ref[...], preferred_element_type=jnp.float32)


_tc_nodes = pl.pallas_call(
    _tc_nodes_body,
    out_shape=(jax.ShapeDtypeStruct((N, H), jnp.float32),
               jax.ShapeDtypeStruct((N, H), jnp.float32)),
)


def _tc_edges_body(ea_ref, wet_ref, be_ref, wc2t_ref, bc_ref, b_ref):
    ee = jnp.maximum(
        jnp.dot(ea_ref[...], wet_ref[...], preferred_element_type=jnp.float32)
        + be_ref[...], 0.0)
    b_ref[...] = (
        jnp.dot(ee, wc2t_ref[...], preferred_element_type=jnp.float32)
        + bc_ref[...])


_tc_edges = pl.pallas_call(
    _tc_edges_body,
    grid=(E // _EDGE_BLK,),
    in_specs=[
        pl.BlockSpec((_EDGE_BLK, D_EDGE), lambda i: (i, 0)),
        pl.BlockSpec((D_EDGE, H), lambda i: (0, 0)),
        pl.BlockSpec((1, H), lambda i: (0, 0)),
        pl.BlockSpec((H, H), lambda i: (0, 0)),
        pl.BlockSpec((1, H), lambda i: (0, 0)),
    ],
    out_specs=pl.BlockSpec((_EDGE_BLK, H), lambda i: (i, 0)),
    out_shape=jax.ShapeDtypeStruct((E, H), jnp.float32),
)


def _tc_combine_body(ne_ref, p_ref, y_ref):
    y = ne_ref[...] + p_ref[0] + p_ref[1]
    y_ref[...] = jnp.concatenate(
        [y, jnp.zeros((N, 128 - H), jnp.float32)], axis=1)


_tc_combine = pl.pallas_call(
    _tc_combine_body,
    out_shape=jax.ShapeDtypeStruct((N, 128), jnp.float32),
)


def _tc_head_body(post_ref, wot_ref, bo_ref, out_ref):
    logits = (jnp.dot(post_ref[...], wot_ref[...],
                      preferred_element_type=jnp.float32) + bo_ref[...])
    out_ref[...] = jax.nn.sigmoid(logits)


_tc_head = pl.pallas_call(
    _tc_head_body,
    out_shape=jax.ShapeDtypeStruct((P, 1), jnp.float32),
)


@functools.partial(
    pl.kernel,
    out_type=jax.ShapeDtypeStruct((NC, N, H), jnp.float32),
    mesh=_mesh,
    compiler_params=pltpu.CompilerParams(use_tc_tiling_on_sc=False),
    scratch_types=[
        pltpu.VMEM((FULL_ROUNDS, S), jnp.int32),
        pltpu.VMEM((FULL_ROUNDS, S), jnp.int32),
        pltpu.VMEM((1, S), jnp.int32),
        pltpu.VMEM((1, S), jnp.int32),
        pltpu.VMEM((S, H), jnp.float32),
        pltpu.VMEM((S, H), jnp.float32),
        pltpu.VMEM((LAST_STRIPE, H), jnp.float32),
        pltpu.VMEM_SHARED((N, H), jnp.float32),
        pltpu.VMEM_SHARED((N, H), jnp.float32),
        pltpu.SemaphoreType.DMA,
        pltpu.SemaphoreType.DMA,
        pltpu.SemaphoreType.DMA,
        pltpu.SemaphoreType.DMA,
        pltpu.SemaphoreType.DMA,
        pltpu.SemaphoreType.DMA,
    ],
)
def _sc_messages(a_hbm, b_hbm, src_hbm, dst_hbm, out_hbm,
                 src_all, dst_all, st_v, dt_v, b0, b1, zbuf_v, msg_sh, a_sh,
                 sb0, sb1, sg0, sg1, ss0, ss1):
    cid = lax.axis_index("c")
    sid = lax.axis_index("s")
    wid = sid * NC + cid

    @pl.when(sid < NS - 1)
    def _a_main():
        off = pl.multiple_of(sid * STRIPE, 8)
        pltpu.sync_copy(a_hbm.at[pl.ds(off, STRIPE), :],
                        a_sh.at[pl.ds(off, STRIPE), :])

    @pl.when(sid == NS - 1)
    def _a_last():
        off = (NS - 1) * STRIPE
        pltpu.sync_copy(a_hbm.at[pl.ds(off, LAST_STRIPE), :],
                        a_sh.at[pl.ds(off, LAST_STRIPE), :])

    def _zb(i, carry):
        zbuf_v[i, pl.ds(0, 16)] = jnp.zeros((16,), jnp.float32)
        zbuf_v[i, pl.ds(16, 16)] = jnp.zeros((16,), jnp.float32)
        return carry

    lax.fori_loop(0, LAST_STRIPE, _zb, 0)

    @pl.when(sid < NS - 1)
    def _z_main():
        off = pl.multiple_of(sid * STRIPE, 8)
        pltpu.sync_copy(zbuf_v.at[pl.ds(0, STRIPE), :],
                        msg_sh.at[pl.ds(off, STRIPE), :])

    @pl.when(sid == NS - 1)
    def _z_last():
        pltpu.sync_copy(zbuf_v, msg_sh.at[pl.ds((NS - 1) * STRIPE,
                                                LAST_STRIPE), :])

    plsc.subcore_barrier()

    base = wid * FULL_ROUNDS
    pltpu.sync_copy(src_hbm.at[pl.ds(base, FULL_ROUNDS), :], src_all)
    pltpu.sync_copy(dst_hbm.at[pl.ds(base, FULL_ROUNDS), :], dst_all)

    def _fire_load(c, buf, sem):
        eb = pl.multiple_of(c * S, 8)
        pltpu.async_copy(b_hbm.at[pl.ds(eb, S), :], buf, sem)

    def _wait_load(buf, sem):
        pltpu.make_async_copy(b_hbm.at[pl.ds(0, S), :], buf, sem).wait()

    _fire_load(base, b0, sb0)
    _fire_load(base + 1, b1, sb1)

    def _body(k, carry):
        c0 = base + 2 * k
        _wait_load(b0, sb0)
        pltpu.async_copy(a_sh.at[src_all.at[2 * k]], b0, sg0, add=True).wait()
        sc0 = pltpu.async_copy(b0, msg_sh.at[dst_all.at[2 * k]], ss0,
                               add=True)
        _wait_load(b1, sb1)
        pltpu.async_copy(a_sh.at[src_all.at[2 * k + 1]], b1, sg1,
                         add=True).wait()
        sc1 = pltpu.async_copy(b1, msg_sh.at[dst_all.at[2 * k + 1]], ss1,
                               add=True)
        sc0.wait()
        _fire_load(jnp.minimum(c0 + 2, NCH - 1), b0, sb0)
        sc1.wait()
        _fire_load(jnp.minimum(c0 + 3, NCH - 1), b1, sb1)
        return carry

    lax.fori_loop(0, FULL_ROUNDS // 2, _body, 0)
    _wait_load(b0, sb0)
    _wait_load(b1, sb1)

    @pl.when(wid < LEFTOVER)
    def _tail():
        c = FULL_ROUNDS * NW + wid
        pltpu.sync_copy(src_hbm.at[pl.ds(c, 1), :], st_v)
        pltpu.sync_copy(dst_hbm.at[pl.ds(c, 1), :], dt_v)
        eb = pl.multiple_of(c * S, 8)
        pltpu.sync_copy(b_hbm.at[pl.ds(eb, S), :], b0)
        pltpu.async_copy(a_sh.at[st_v.at[0]], b0, sg0, add=True).wait()
        pltpu.sync_copy(b0, msg_sh.at[dt_v.at[0]], add=True)

    plsc.subcore_barrier()

    @pl.when(sid < NS - 1)
    def _d_main():
        off = pl.multiple_of(sid * STRIPE, 8)
        pltpu.sync_copy(msg_sh.at[pl.ds(off, STRIPE), :],
                        out_hbm.at[cid, pl.ds(off, STRIPE), :])

    @pl.when(sid == NS - 1)
    def _d_last():
        pltpu.sync_copy(msg_sh.at[pl.ds((NS - 1) * STRIPE, LAST_STRIPE), :],
                        out_hbm.at[cid, pl.ds((NS - 1) * STRIPE,
                                              LAST_STRIPE), :])


@functools.partial(
    pl.kernel,
    out_type=jax.ShapeDtypeStruct((P_PAD, 128), jnp.float32),
    mesh=_mesh,
    compiler_params=pltpu.CompilerParams(use_tc_tiling_on_sc=False),
    scratch_types=[
        pltpu.VMEM((S,), jnp.int32),
        pltpu.VMEM((S, 128), jnp.float32),
        pltpu.SemaphoreType.DMA,
    ],
)
def _sc_post_gather(y_hbm, pm_hbm, out_hbm, pm_v, rows_v, sem):
    cid = lax.axis_index("c")
    sid = lax.axis_index("s")
    wid = sid * NC + cid

    def _do(chunk):
        off = pl.multiple_of(chunk * S, 8)
        pltpu.sync_copy(pm_hbm.at[pl.ds(off, S)], pm_v)
        pltpu.async_copy(y_hbm.at[pm_v], rows_v, sem).wait()
        pltpu.sync_copy(rows_v, out_hbm.at[pl.ds(off, S), :])

    _do(wid)

    @pl.when(wid < NCHUNK_P - NW)
    def _tail():
        _do(wid + NW)


def kernel(node_features, edge_index, edge_attr, post_mask,
           W_node, b_node, W_edge, b_edge, W_conv, b_conv, W_out, b_out):
    wnt = W_node.T
    wet = W_edge.T
    wc1t = W_conv[:, :H].T
    wc2t = W_conv[:, H:].T
    wot = W_out.T
    bn = b_node.reshape(1, H)
    be = b_edge.reshape(1, H)
    bc = b_conv.reshape(1, H)
    bo = b_out.reshape(1, 1)

    node_emb, a_rows = _tc_nodes(node_features, wnt, bn, wc1t)
    b_rows = _tc_edges(edge_attr, wet, be, wc2t, bc)

    src2 = edge_index[0].reshape(NCH, S)
    dst2 = edge_index[1].reshape(NCH, S)
    partials = _sc_messages(a_rows, b_rows, src2, dst2)

    y = _tc_combine(node_emb, partials)
    pm_pad = jnp.concatenate(
        [post_mask, jnp.zeros((P_PAD - P,), dtype=post_mask.dtype)])
    post_full = _sc_post_gather(y, pm_pad)
    out = _tc_head(post_full[:P, :H], wot, bo)
    return out.reshape(P)
